# Initial kernel scaffold; baseline (speedup 1.0000x reference)
import functools

import jax
import jax.numpy as jnp
from jax import lax
from jax.experimental import pallas as pl
from jax.experimental.pallas import tpu as pltpu
from jax.experimental.pallas import tpu_sc as plsc

_VOCAB = 100000
_EMB = 30
_B = 16384
_LT = 20
_LD = 200
_DP = 32
_SEQ = 2 * (_LT + _LD)
_NC = 2
_NS = 16
_NW = _NC * _NS
_ROWS_W = _B // _NW
_R = 2
_NCHUNK = _ROWS_W // _R
_IDXC = _R * _SEQ
_U = 20
_FIELDS = ((0, _LT), (_LT, _LD), (_LT + _LD, _LT), (2 * _LT + _LD, _LD))


def _sc_pool(tbl, idx):
    mesh = plsc.VectorSubcoreMesh(core_axis_name="c", subcore_axis_name="s")

    @functools.partial(
        pl.kernel,
        out_type=jax.ShapeDtypeStruct((_B, 4 * _DP), jnp.float32),
        mesh=mesh,
        scratch_types=[
            pltpu.VMEM((2, _IDXC), jnp.int32),
            pltpu.VMEM((_IDXC, _DP), jnp.float32),
            pltpu.VMEM((_IDXC, _DP), jnp.float32),
            pltpu.VMEM((2, _R, 4 * _DP), jnp.float32),
            pltpu.SemaphoreType.DMA,
            pltpu.SemaphoreType.DMA,
            pltpu.SemaphoreType.DMA,
            pltpu.SemaphoreType.DMA,
            pltpu.SemaphoreType.DMA,
            pltpu.SemaphoreType.DMA,
        ],
    )
    def k(tbl_hbm, idx_hbm, out_hbm, idx_v, rows0, rows1, outb,
          sg0, sg1, si0, si1, so0, so1):
        wid = lax.axis_index("s") * _NC + lax.axis_index("c")
        row0 = wid * _ROWS_W
        rows = (rows0, rows1)
        sg = (sg0, sg1)
        si = (si0, si1)
        so = (so0, so1)

        def idx_src(c):
            return idx_hbm.at[pl.ds((row0 + c * _R) * _SEQ, _IDXC)]

        pltpu.sync_copy(idx_src(0), idx_v.at[0])
        pltpu.async_copy(tbl_hbm.at[idx_v.at[0]], rows[0], sg[0])
        pltpu.async_copy(idx_src(1), idx_v.at[1], si[1])

        @pl.loop(0, _NCHUNK, step=2)
        def _(c):
            for b in range(2):
                cc = c + b
                nb = 1 - b

                @pl.when(cc + 1 < _NCHUNK)
                def _():
                    pltpu.make_async_copy(idx_src(cc + 1), idx_v.at[nb],
                                          si[nb]).wait()
                    pltpu.async_copy(tbl_hbm.at[idx_v.at[nb]], rows[nb], sg[nb])

                pltpu.make_async_copy(tbl_hbm.at[idx_v.at[b]], rows[b],
                                      sg[b]).wait()

                @pl.when(cc + 2 < _NCHUNK)
                def _():
                    pltpu.async_copy(idx_src(cc + 2), idx_v.at[b], si[b])

                @pl.when(cc >= 2)
                def _():
                    pltpu.make_async_copy(outb.at[b],
                                          out_hbm.at[pl.ds(row0, _R)],
                                          so[b]).wait()

                rb = rows[b]
                for r in range(_R):
                    for fi, (off, L) in enumerate(_FIELDS):
                        base = r * _SEQ + off

                        def red(i, acc, base=base):
                            a0, a1 = acc
                            for u in range(_U):
                                j = base + i * _U + u
                                a0 = a0 + rb[j, pl.ds(0, 16)]
                                a1 = a1 + rb[j, pl.ds(16, 16)]
                            return (a0, a1)

                        z = jnp.zeros((16,), jnp.float32)
                        a0, a1 = lax.fori_loop(0, L // _U, red, (z, z))
                        outb[b, r, pl.ds(fi * _DP, 16)] = a0
                        outb[b, r, pl.ds(fi * _DP + 16, 16)] = a1

                pltpu.async_copy(outb.at[b],
                                 out_hbm.at[pl.ds(row0 + cc * _R, _R)], so[b])

        for b in range(2):
            pltpu.make_async_copy(outb.at[b], out_hbm.at[pl.ds(row0, _R)],
                                  so[b]).wait()

    return k(tbl, idx)


def _mlp(x, w1p, b1p, w2p, b2p, w3p, b3p):
    blk = 2048

    def body(x_ref, w1_ref, b1_ref, w2_ref, b2_ref, w3_ref, b3_ref, o_ref):
        h = jnp.maximum(x_ref[...], 0.0)
        h = jnp.dot(h, w1_ref[...], preferred_element_type=jnp.float32)
        h = jnp.maximum(h + b1_ref[...], 0.0)
        h = jnp.dot(h, w2_ref[...], preferred_element_type=jnp.float32)
        h = jnp.maximum(h + b2_ref[...], 0.0)
        z = jnp.dot(h, w3_ref[...], preferred_element_type=jnp.float32)
        z = z + b3_ref[...]
        o_ref[...] = jax.nn.sigmoid(z[:, :1])

    wspec = pl.BlockSpec((128, 128), lambda i: (0, 0))
    bspec = pl.BlockSpec((1, 128), lambda i: (0, 0))
    return pl.pallas_call(
        body,
        grid=(_B // blk,),
        in_specs=[pl.BlockSpec((blk, 128), lambda i: (i, 0)),
                  wspec, bspec, wspec, bspec, wspec, bspec],
        out_specs=pl.BlockSpec((blk, 1), lambda i: (i, 0)),
        out_shape=jax.ShapeDtypeStruct((_B, 1), jnp.float32),
    )(x, w1p, b1p, w2p, b2p, w3p, b3p)


def kernel(content_title, content_description, topic_title, topic_description,
           E_ct, E_cd, E_tt, E_td, W1, b1, W2, b2, W3, b3):
    tbl = jnp.concatenate([E_ct, E_cd, E_tt, E_td], axis=0)
    tbl = jnp.pad(tbl, ((0, 0), (0, _DP - _EMB)))
    idx = jnp.concatenate(
        [content_title.astype(jnp.int32),
         content_description.astype(jnp.int32) + _VOCAB,
         topic_title.astype(jnp.int32) + 2 * _VOCAB,
         topic_description.astype(jnp.int32) + 3 * _VOCAB],
        axis=1).reshape(-1)

    pooled = _sc_pool(tbl, idx)

    w1p = jnp.pad(W1.reshape(4, _EMB, 30),
                  ((0, 0), (0, _DP - _EMB), (0, 98))).reshape(4 * _DP, 128)
    b1p = jnp.pad(b1, (0, 98)).reshape(1, 128)
    w2p = jnp.pad(W2, ((0, 98), (0, 98)))
    b2p = jnp.pad(b2, (0, 98)).reshape(1, 128)
    w3p = jnp.pad(W3, ((0, 98), (0, 127)))
    b3p = jnp.pad(b3, (0, 127)).reshape(1, 128)

    return _mlp(pooled, w1p, b1p, w2p, b2p, w3p, b3p)


# speedup vs baseline: 32.5871x; 32.5871x over previous
import functools

import jax
import jax.numpy as jnp
from jax import lax
from jax.experimental import pallas as pl
from jax.experimental.pallas import tpu as pltpu
from jax.experimental.pallas import tpu_sc as plsc

_VOCAB = 100000
_EMB = 30
_B = 16384
_LT = 20
_LD = 200
_DP = 32
_SEQ = 2 * (_LT + _LD)
_NC = 2
_NS = 16
_NW = _NC * _NS
_ROWS_W = _B // _NW
_R = 2
_NCHUNK = _ROWS_W // _R
_IDXC = _R * _SEQ
_U = 20
_FIELDS = ((0, _LT), (_LT, _LD), (_LT + _LD, _LT), (2 * _LT + _LD, _LD))


def _sc_pool(tbl, idx):
    mesh = plsc.VectorSubcoreMesh(core_axis_name="c", subcore_axis_name="s")

    @functools.partial(
        pl.kernel,
        out_type=jax.ShapeDtypeStruct((_B, 4 * _DP), jnp.float32),
        mesh=mesh,
        compiler_params=pltpu.CompilerParams(use_tc_tiling_on_sc=False),
        scratch_types=[
            pltpu.VMEM((_IDXC,), jnp.int32),
            pltpu.VMEM((_IDXC,), jnp.int32),
            pltpu.VMEM((_IDXC, _DP), jnp.float32),
            pltpu.VMEM((_IDXC, _DP), jnp.float32),
            pltpu.VMEM((2, _R, 4 * _DP), jnp.float32),
            pltpu.SemaphoreType.DMA,
            pltpu.SemaphoreType.DMA,
            pltpu.SemaphoreType.DMA,
            pltpu.SemaphoreType.DMA,
            pltpu.SemaphoreType.DMA,
            pltpu.SemaphoreType.DMA,
        ],
    )
    def k(tbl_hbm, idx_hbm, out_hbm, idx0, idx1, rows0, rows1, outb,
          sg0, sg1, si0, si1, so0, so1):
        wid = lax.axis_index("s") * _NC + lax.axis_index("c")
        row0 = wid * _ROWS_W
        idx_v = (idx0, idx1)
        rows = (rows0, rows1)
        sg = (sg0, sg1)
        si = (si0, si1)
        so = (so0, so1)

        def idx_src(c):
            return idx_hbm.at[pl.ds((row0 + c * _R) * _SEQ, _IDXC)]

        pltpu.sync_copy(idx_src(0), idx_v[0])
        pltpu.async_copy(tbl_hbm.at[idx_v[0]], rows[0], sg[0])
        pltpu.async_copy(idx_src(1), idx_v[1], si[1])

        @pl.loop(0, _NCHUNK, step=2)
        def _(c):
            for b in range(2):
                cc = c + b
                nb = 1 - b

                @pl.when(cc + 1 < _NCHUNK)
                def _():
                    pltpu.make_async_copy(idx_src(cc + 1), idx_v[nb],
                                          si[nb]).wait()
                    pltpu.async_copy(tbl_hbm.at[idx_v[nb]], rows[nb], sg[nb])

                pltpu.make_async_copy(tbl_hbm.at[idx_v[b]], rows[b],
                                      sg[b]).wait()

                @pl.when(cc + 2 < _NCHUNK)
                def _():
                    pltpu.async_copy(idx_src(cc + 2), idx_v[b], si[b])

                @pl.when(cc >= 2)
                def _():
                    pltpu.make_async_copy(outb.at[b],
                                          out_hbm.at[pl.ds(row0, _R)],
                                          so[b]).wait()

                rb = rows[b]
                for r in range(_R):
                    for fi, (off, L) in enumerate(_FIELDS):
                        base = r * _SEQ + off

                        def red(i, acc, base=base):
                            a0, a1 = acc
                            for u in range(_U):
                                j = base + i * _U + u
                                a0 = a0 + rb[j, pl.ds(0, 16)]
                                a1 = a1 + rb[j, pl.ds(16, 16)]
                            return (a0, a1)

                        z = jnp.zeros((16,), jnp.float32)
                        a0, a1 = lax.fori_loop(0, L // _U, red, (z, z))
                        outb[b, r, pl.ds(fi * _DP, 16)] = a0
                        outb[b, r, pl.ds(fi * _DP + 16, 16)] = a1

                pltpu.async_copy(outb.at[b],
                                 out_hbm.at[pl.ds(row0 + cc * _R, _R)], so[b])

        for b in range(2):
            pltpu.make_async_copy(outb.at[b], out_hbm.at[pl.ds(row0, _R)],
                                  so[b]).wait()

    return k(tbl, idx)


def _mlp(x, w1p, b1p, w2p, b2p, w3p, b3p):
    blk = 2048

    def body(x_ref, w1_ref, b1_ref, w2_ref, b2_ref, w3_ref, b3_ref, o_ref):
        h = jnp.maximum(x_ref[...], 0.0)
        h = jnp.dot(h, w1_ref[...], preferred_element_type=jnp.float32)
        h = jnp.maximum(h + b1_ref[...], 0.0)
        h = jnp.dot(h, w2_ref[...], preferred_element_type=jnp.float32)
        h = jnp.maximum(h + b2_ref[...], 0.0)
        z = jnp.dot(h, w3_ref[...], preferred_element_type=jnp.float32)
        z = z + b3_ref[...]
        o_ref[...] = jax.nn.sigmoid(z[:, :1])

    wspec = pl.BlockSpec((128, 128), lambda i: (0, 0))
    bspec = pl.BlockSpec((1, 128), lambda i: (0, 0))
    return pl.pallas_call(
        body,
        grid=(_B // blk,),
        in_specs=[pl.BlockSpec((blk, 128), lambda i: (i, 0)),
                  wspec, bspec, wspec, bspec, wspec, bspec],
        out_specs=pl.BlockSpec((blk, 1), lambda i: (i, 0)),
        out_shape=jax.ShapeDtypeStruct((_B, 1), jnp.float32),
    )(x, w1p, b1p, w2p, b2p, w3p, b3p)


def kernel(content_title, content_description, topic_title, topic_description,
           E_ct, E_cd, E_tt, E_td, W1, b1, W2, b2, W3, b3):
    tbl = jnp.concatenate([E_ct, E_cd, E_tt, E_td], axis=0)
    tbl = jnp.pad(tbl, ((0, 0), (0, _DP - _EMB)))
    idx = jnp.concatenate(
        [content_title.astype(jnp.int32),
         content_description.astype(jnp.int32) + _VOCAB,
         topic_title.astype(jnp.int32) + 2 * _VOCAB,
         topic_description.astype(jnp.int32) + 3 * _VOCAB],
        axis=1).reshape(-1)

    pooled = _sc_pool(tbl, idx)

    w1p = jnp.pad(W1.reshape(4, _EMB, 30),
                  ((0, 0), (0, _DP - _EMB), (0, 98))).reshape(4 * _DP, 128)
    b1p = jnp.pad(b1, (0, 98)).reshape(1, 128)
    w2p = jnp.pad(W2, ((0, 98), (0, 98)))
    b2p = jnp.pad(b2, (0, 98)).reshape(1, 128)
    w3p = jnp.pad(W3, ((0, 98), (0, 127)))
    b3p = jnp.pad(b3, (0, 127)).reshape(1, 128)

    return _mlp(pooled, w1p, b1p, w2p, b2p, w3p, b3p)


# speedup vs baseline: 33.9424x; 1.0416x over previous
import functools

import jax
import jax.numpy as jnp
from jax import lax
from jax.experimental import pallas as pl
from jax.experimental.pallas import tpu as pltpu
from jax.experimental.pallas import tpu_sc as plsc

_VOCAB = 100000
_EMB = 30
_B = 16384
_LT = 20
_LD = 200
_DP = 32
_SEQ = 2 * (_LT + _LD)
_NC = 2
_NS = 16
_NW = _NC * _NS
_ROWS_W = _B // _NW
_R = 4
_NCHUNK = _ROWS_W // _R
_IDXC = _R * _SEQ
_U = 20
_FIELDS = ((0, _LT), (_LT, _LD), (_LT + _LD, _LT), (2 * _LT + _LD, _LD))


def _sc_pool(tbl, idx):
    mesh = plsc.VectorSubcoreMesh(core_axis_name="c", subcore_axis_name="s")

    @functools.partial(
        pl.kernel,
        out_type=jax.ShapeDtypeStruct((_B, 4 * _DP), jnp.bfloat16),
        mesh=mesh,
        compiler_params=pltpu.CompilerParams(use_tc_tiling_on_sc=False),
        scratch_types=[
            pltpu.VMEM((_IDXC,), jnp.int32),
            pltpu.VMEM((_IDXC,), jnp.int32),
            pltpu.VMEM((_IDXC, _DP), jnp.bfloat16),
            pltpu.VMEM((_IDXC, _DP), jnp.bfloat16),
            pltpu.VMEM((2, _R, 4 * _DP), jnp.bfloat16),
            pltpu.SemaphoreType.DMA,
            pltpu.SemaphoreType.DMA,
            pltpu.SemaphoreType.DMA,
            pltpu.SemaphoreType.DMA,
            pltpu.SemaphoreType.DMA,
            pltpu.SemaphoreType.DMA,
        ],
    )
    def k(tbl_hbm, idx_hbm, out_hbm, idx0, idx1, rows0, rows1, outb,
          sg0, sg1, si0, si1, so0, so1):
        wid = lax.axis_index("s") * _NC + lax.axis_index("c")
        row0 = wid * _ROWS_W
        idx_v = (idx0, idx1)
        rows = (rows0, rows1)
        sg = (sg0, sg1)
        si = (si0, si1)
        so = (so0, so1)

        def idx_src(c):
            return idx_hbm.at[pl.ds((row0 + c * _R) * _SEQ, _IDXC)]

        pltpu.sync_copy(idx_src(0), idx_v[0])
        pltpu.async_copy(tbl_hbm.at[idx_v[0]], rows[0], sg[0])
        pltpu.async_copy(idx_src(1), idx_v[1], si[1])

        @pl.loop(0, _NCHUNK, step=2)
        def _(c):
            for b in range(2):
                cc = c + b
                nb = 1 - b

                @pl.when(cc + 1 < _NCHUNK)
                def _():
                    pltpu.make_async_copy(idx_src(cc + 1), idx_v[nb],
                                          si[nb]).wait()
                    pltpu.async_copy(tbl_hbm.at[idx_v[nb]], rows[nb], sg[nb])

                pltpu.make_async_copy(tbl_hbm.at[idx_v[b]], rows[b],
                                      sg[b]).wait()

                @pl.when(cc + 2 < _NCHUNK)
                def _():
                    pltpu.async_copy(idx_src(cc + 2), idx_v[b], si[b])

                @pl.when(cc >= 2)
                def _():
                    pltpu.make_async_copy(outb.at[b],
                                          out_hbm.at[pl.ds(row0, _R)],
                                          so[b]).wait()

                rb = rows[b]
                for r in range(_R):
                    for fi, (off, L) in enumerate(_FIELDS):
                        base = r * _SEQ + off

                        def red(i, acc, base=base):
                            for u in range(_U):
                                j = base + i * _U + u
                                acc = acc + rb[j, :]
                            return acc

                        z = jnp.zeros((_DP,), jnp.bfloat16)
                        a = lax.fori_loop(0, L // _U, red, z)
                        outb[b, r, pl.ds(fi * _DP, _DP)] = a

                pltpu.async_copy(outb.at[b],
                                 out_hbm.at[pl.ds(row0 + cc * _R, _R)], so[b])

        for b in range(2):
            pltpu.make_async_copy(outb.at[b], out_hbm.at[pl.ds(row0, _R)],
                                  so[b]).wait()

    return k(tbl, idx)


def _mlp(x, w1p, b1p, w2p, b2p, w3p, b3p):
    blk = 2048

    def body(x_ref, w1_ref, b1_ref, w2_ref, b2_ref, w3_ref, b3_ref, o_ref):
        h = jnp.maximum(x_ref[...].astype(jnp.float32), 0.0)
        h = jnp.dot(h, w1_ref[...], preferred_element_type=jnp.float32)
        h = jnp.maximum(h + b1_ref[...], 0.0)
        h = jnp.dot(h, w2_ref[...], preferred_element_type=jnp.float32)
        h = jnp.maximum(h + b2_ref[...], 0.0)
        z = jnp.dot(h, w3_ref[...], preferred_element_type=jnp.float32)
        z = z + b3_ref[...]
        o_ref[...] = jax.nn.sigmoid(z[:, :1])

    wspec = pl.BlockSpec((128, 128), lambda i: (0, 0))
    bspec = pl.BlockSpec((1, 128), lambda i: (0, 0))
    return pl.pallas_call(
        body,
        grid=(_B // blk,),
        in_specs=[pl.BlockSpec((blk, 128), lambda i: (i, 0)),
                  wspec, bspec, wspec, bspec, wspec, bspec],
        out_specs=pl.BlockSpec((blk, 1), lambda i: (i, 0)),
        out_shape=jax.ShapeDtypeStruct((_B, 1), jnp.float32),
    )(x, w1p, b1p, w2p, b2p, w3p, b3p)


def kernel(content_title, content_description, topic_title, topic_description,
           E_ct, E_cd, E_tt, E_td, W1, b1, W2, b2, W3, b3):
    tbl = jnp.concatenate([E_ct, E_cd, E_tt, E_td], axis=0)
    tbl = jnp.pad(tbl, ((0, 0), (0, _DP - _EMB))).astype(jnp.bfloat16)
    idx = jnp.concatenate(
        [content_title.astype(jnp.int32),
         content_description.astype(jnp.int32) + _VOCAB,
         topic_title.astype(jnp.int32) + 2 * _VOCAB,
         topic_description.astype(jnp.int32) + 3 * _VOCAB],
        axis=1).reshape(-1)

    pooled = _sc_pool(tbl, idx)

    w1p = jnp.pad(W1.reshape(4, _EMB, 30),
                  ((0, 0), (0, _DP - _EMB), (0, 98))).reshape(4 * _DP, 128)
    b1p = jnp.pad(b1, (0, 98)).reshape(1, 128)
    w2p = jnp.pad(W2, ((0, 98), (0, 98)))
    b2p = jnp.pad(b2, (0, 98)).reshape(1, 128)
    w3p = jnp.pad(W3, ((0, 98), (0, 127)))
    b3p = jnp.pad(b3, (0, 127)).reshape(1, 128)

    return _mlp(pooled, w1p, b1p, w2p, b2p, w3p, b3p)
